# baseline (speedup 1.0000x reference)
import functools

import jax
import jax.numpy as jnp
from jax import lax
from jax.experimental import pallas as pl
from jax.experimental.pallas import tpu as pltpu
from jax.experimental.pallas import tpu_sc as plsc

_K = 64
_L = 16
_NBINS = 256
_ROWS = 128
_N = 32768
_NCHUNK = _N // _L
_UNROLL = 8

_SIGN = -2147483648
_M31 = 0x7FFFFFFF


def _ukey(v, c31):
    b = plsc.bitcast(v, jnp.int32)
    return b ^ (lax.shift_right_arithmetic(b, c31) | jnp.int32(_SIGN))


def _key_to_f32(t_s16):
    bits = jnp.where(t_s16 < 0, t_s16 ^ jnp.int32(_M31), t_s16)
    return plsc.bitcast(bits, jnp.float32)


def _sc_body(x_hbm, o_hbm, xv0, xv1, xv2, hist, sem0, sem1, sem2):
    nc = 2
    wid = lax.axis_index("s") * nc + lax.axis_index("c")
    bufs = (xv0, xv1, xv2)
    sems = (sem0, sem1, sem2)
    lane = lax.iota(jnp.int32, _L)
    spread = (lane & jnp.int32(3)) * jnp.int32(_NBINS)
    c31 = jnp.full((_L,), 31, jnp.int32)
    ones = jnp.ones((_L,), jnp.int32)
    zeros16i = jnp.zeros((_L,), jnp.int32)
    zeros16f = jnp.zeros((_L,), jnp.float32)
    rev_lane = jnp.int32(_L - 1) - lane
    shift_vecs = {s: jnp.full((_L,), s, jnp.int32) for s in (8, 16, 24)}

    def run_round(xv, shift, prefix, need):
        for z in range((4 * _NBINS) // _L):
            hist[pl.ds(z * _L, _L)] = zeros16i

        if shift == 24:
            @plsc.parallel_loop(0, _NCHUNK, 1, unroll=_UNROLL)
            def _hpass(i):
                v = xv[pl.ds(i * _L, _L)]
                u = _ukey(v, c31)
                dig = lax.shift_right_logical(u, shift_vecs[24])
                plsc.addupdate_scatter(hist, [dig + spread], ones)
        else:
            msk_shift = shift_vecs[shift + 8]

            @plsc.parallel_loop(0, _NCHUNK, 1, unroll=_UNROLL)
            def _hpass(i):
                v = xv[pl.ds(i * _L, _L)]
                u = _ukey(v, c31)
                m = lax.shift_right_logical(u, msk_shift) == prefix
                if shift == 0:
                    dig = u & jnp.int32(0xFF)
                else:
                    dig = (lax.shift_right_logical(u, shift_vecs[shift])
                           & jnp.int32(0xFF))
                plsc.addupdate_scatter(hist, [dig + spread], ones, mask=m)

        def scan(i, carry):
            run, dstar = carry
            cc = (_NBINS // _L - 1) - i
            vv = (hist[pl.ds(cc * _L, _L)]
                  + hist[pl.ds(_NBINS + cc * _L, _L)]
                  + hist[pl.ds(2 * _NBINS + cc * _L, _L)]
                  + hist[pl.ds(3 * _NBINS + cc * _L, _L)])
            rv = lax.rev(vv, dimensions=(0,))
            sfx = plsc.cumsum(rv) + run
            digs_desc = jnp.int32(cc * _L) + rev_lane
            cand = jnp.where(sfx >= need, digs_desc, jnp.int32(-1))
            return jnp.max(sfx), jnp.maximum(dstar, jnp.max(cand))

        _, dstar = lax.fori_loop(0, _NBINS // _L, scan,
                                 (jnp.int32(0), jnp.int32(-1)))

        def counts(c, carry):
            cgt, sge = carry
            digs = jnp.int32(c * _L) + lane
            vv = (hist[pl.ds(c * _L, _L)]
                  + hist[pl.ds(_NBINS + c * _L, _L)]
                  + hist[pl.ds(2 * _NBINS + c * _L, _L)]
                  + hist[pl.ds(3 * _NBINS + c * _L, _L)])
            cgt = cgt + jnp.sum(jnp.where(digs > dstar, vv, 0))
            sge = sge + jnp.sum(jnp.where(digs >= dstar, vv, 0))
            return cgt, sge

        cgt, sge = lax.fori_loop(0, _NBINS // _L, counts,
                                 (jnp.int32(0), jnp.int32(0)))
        return dstar, cgt, sge

    def resolve(xv, shift, prefix, need):
        dstar, cgt, sge = run_round(xv, shift, prefix, need)
        prefix2 = prefix * jnp.int32(_NBINS) + dstar
        need2 = need - cgt
        m = sge - cgt
        if shift == 0:
            return prefix2, need2, m

        def exit_fn(op):
            p2, n2 = op
            t_u = p2 * jnp.int32(1 << shift)
            return t_u, n2, n2

        def cont_fn(op):
            p2, n2 = op
            return resolve(xv, shift - 8, p2, n2)

        return lax.cond(m == need2, exit_fn, cont_fn, (prefix2, need2))

    def process(xv):
        t_u, r, total_eq = resolve(xv, 24, jnp.int32(0), jnp.int32(_K))
        t_s = t_u ^ jnp.int32(_SIGN)
        tf = _key_to_f32(jnp.full((_L,), 0, jnp.int32) + t_s)

        def simple_pass(_o):
            @plsc.parallel_loop(0, _NCHUNK, 1, unroll=_UNROLL)
            def _body(i):
                v = xv[pl.ds(i * _L, _L)]
                xv[pl.ds(i * _L, _L)] = jnp.where(v >= tf, v, zeros16f)
            return 0

        def tie_pass(_o):
            def body(i, carry):
                v = xv[pl.ds(i * _L, _L)]
                gt = v > tf
                eq = v == tf
                pc = plsc.cumsum(eq.astype(jnp.int32)) + carry
                keep = gt | (eq & (pc <= r))
                xv[pl.ds(i * _L, _L)] = jnp.where(keep, v, zeros16f)
                return jnp.max(pc)
            lax.fori_loop(0, _NCHUNK, body, jnp.int32(0))
            return 0

        lax.cond(total_eq == r, simple_pass, tie_pass, 0)

    nrows = _ROWS // 32
    base = wid * nrows
    in_handles = [None] * nrows
    pending_out = [None] * len(bufs)

    in_handles[0] = pltpu.make_async_copy(x_hbm.at[base], bufs[0], sems[0])
    in_handles[0].start()
    for j in range(nrows):
        b = j % len(bufs)
        in_handles[j].wait()
        if j + 1 < nrows:
            nb = (j + 1) % len(bufs)
            if pending_out[nb] is not None:
                pending_out[nb].wait()
                pending_out[nb] = None
            in_handles[j + 1] = pltpu.make_async_copy(
                x_hbm.at[base + (j + 1)], bufs[nb], sems[nb])
            in_handles[j + 1].start()
        process(bufs[b])
        pending_out[b] = pltpu.make_async_copy(
            bufs[b], o_hbm.at[base + j], sems[b])
        pending_out[b].start()
    for h in pending_out:
        if h is not None:
            h.wait()


def kernel(x):
    mesh = plsc.VectorSubcoreMesh(core_axis_name="c", subcore_axis_name="s")
    f = functools.partial(
        pl.kernel,
        out_type=jax.ShapeDtypeStruct((_ROWS, _N), jnp.float32),
        mesh=mesh,
        compiler_params=pltpu.CompilerParams(needs_layout_passes=False),
        scratch_types=[
            pltpu.VMEM((_N,), jnp.float32),
            pltpu.VMEM((_N,), jnp.float32),
            pltpu.VMEM((_N,), jnp.float32),
            pltpu.VMEM((4 * _NBINS,), jnp.int32),
            pltpu.SemaphoreType.DMA,
            pltpu.SemaphoreType.DMA,
            pltpu.SemaphoreType.DMA,
        ],
    )(_sc_body)
    return f(x)


# speedup vs baseline: 1.0905x; 1.0905x over previous
import functools

import jax
import jax.numpy as jnp
from jax import lax
from jax.experimental import pallas as pl
from jax.experimental.pallas import tpu as pltpu
from jax.experimental.pallas import tpu_sc as plsc

_K = 64
_L = 16
_NBINS = 256
_ROWS = 128
_N = 32768
_NCHUNK = _N // _L
_UNROLL = 8

_SIGN = -2147483648
_M31 = 0x7FFFFFFF


def _ukey(v, c31):
    b = plsc.bitcast(v, jnp.int32)
    return b ^ (lax.shift_right_arithmetic(b, c31) | jnp.int32(_SIGN))


def _key_to_f32(t_s16):
    bits = jnp.where(t_s16 < 0, t_s16 ^ jnp.int32(_M31), t_s16)
    return plsc.bitcast(bits, jnp.float32)


def _sc_body(x_hbm, o_hbm, xv0, xv1, xv2, hist, sufb, sem0, sem1, sem2):
    nc = 2
    wid = lax.axis_index("s") * nc + lax.axis_index("c")
    bufs = (xv0, xv1, xv2)
    sems = (sem0, sem1, sem2)
    lane = lax.iota(jnp.int32, _L)
    c31 = jnp.full((_L,), 31, jnp.int32)
    ones = jnp.ones((_L,), jnp.int32)
    zeros16i = jnp.zeros((_L,), jnp.int32)
    zeros16f = jnp.zeros((_L,), jnp.float32)
    rev_lane = jnp.int32(_L - 1) - lane
    shift_vecs = {s: jnp.full((_L,), s, jnp.int32) for s in (8, 16, 24)}

    def run_round(xv, shift, prefix, need):
        for z in range(_NBINS // _L):
            hist[pl.ds(z * _L, _L)] = zeros16i

        if shift == 24:
            @plsc.parallel_loop(0, _NCHUNK, 1, unroll=_UNROLL)
            def _hpass(i):
                v = xv[pl.ds(i * _L, _L)]
                u = _ukey(v, c31)
                dig = lax.shift_right_logical(u, shift_vecs[24])
                plsc.addupdate_scatter(hist, [dig], ones)
        else:
            msk_shift = shift_vecs[shift + 8]

            @plsc.parallel_loop(0, _NCHUNK, 1, unroll=_UNROLL)
            def _hpass(i):
                v = xv[pl.ds(i * _L, _L)]
                u = _ukey(v, c31)
                m = lax.shift_right_logical(u, msk_shift) == prefix
                if shift == 0:
                    dig = u & jnp.int32(0xFF)
                else:
                    dig = (lax.shift_right_logical(u, shift_vecs[shift])
                           & jnp.int32(0xFF))
                plsc.addupdate_scatter(hist, [dig], ones, mask=m)

        def scan(i, carry):
            run, dstar = carry
            cc = (_NBINS // _L - 1) - i
            vv = hist[pl.ds(cc * _L, _L)]
            rv = lax.rev(vv, dimensions=(0,))
            sfx = plsc.cumsum(rv) + run
            sufb[pl.ds(i * _L, _L)] = sfx
            ge = sfx >= need
            pc = plsc.all_reduce_population_count(ge)
            dstar = jnp.maximum(
                dstar, jnp.where(pc > 0, jnp.int32(cc * _L) + pc - 1,
                                 jnp.int32(-1)))
            return jnp.max(sfx), dstar

        _, dstar = lax.fori_loop(
            0, _NBINS // _L, scan,
            (jnp.int32(0), jnp.full((_L,), -1, jnp.int32)))

        sge = plsc.load_gather(sufb, [jnp.int32(_NBINS - 1) - dstar])
        top = dstar >= jnp.int32(_NBINS - 1)
        idx_gt = jnp.where(top, jnp.int32(0), jnp.int32(_NBINS - 2) - dstar)
        cgt = jnp.where(top, jnp.int32(0),
                        plsc.load_gather(sufb, [idx_gt]))
        return dstar, cgt, sge

    def resolve(xv, shift, prefix, need):
        dstar, cgt, sge = run_round(xv, shift, prefix, need)
        prefix2 = prefix * jnp.int32(_NBINS) + dstar
        need2 = need - cgt
        m = sge - cgt
        if shift == 0:
            return prefix2, need2, m

        def exit_fn(op):
            p2, n2 = op
            t_u = p2 * jnp.int32(1 << shift)
            return t_u, n2, n2

        def cont_fn(op):
            p2, n2 = op
            return resolve(xv, shift - 8, p2, n2)

        return lax.cond(jnp.all(m == need2), exit_fn, cont_fn,
                        (prefix2, need2))

    def process(xv):
        t_u, r, total_eq = resolve(
            xv, 24, jnp.zeros((_L,), jnp.int32),
            jnp.full((_L,), _K, jnp.int32))
        t_s = t_u ^ jnp.int32(_SIGN)
        tf = _key_to_f32(t_s)

        def simple_pass(_o):
            @plsc.parallel_loop(0, _NCHUNK, 1, unroll=_UNROLL)
            def _body(i):
                v = xv[pl.ds(i * _L, _L)]
                xv[pl.ds(i * _L, _L)] = jnp.where(v >= tf, v, zeros16f)
            return 0

        def tie_pass(_o):
            def body(i, carry):
                v = xv[pl.ds(i * _L, _L)]
                gt = v > tf
                eq = v == tf
                pc = plsc.cumsum(eq.astype(jnp.int32)) + carry
                keep = gt | (eq & (pc <= r))
                xv[pl.ds(i * _L, _L)] = jnp.where(keep, v, zeros16f)
                return jnp.max(pc)
            lax.fori_loop(0, _NCHUNK, body, jnp.int32(0))
            return 0

        lax.cond(jnp.all(total_eq == r), simple_pass, tie_pass, 0)

    nrows = _ROWS // 32
    base = wid * nrows
    in_handles = [None] * nrows
    pending_out = [None] * len(bufs)

    in_handles[0] = pltpu.make_async_copy(x_hbm.at[base], bufs[0], sems[0])
    in_handles[0].start()
    for j in range(nrows):
        b = j % len(bufs)
        in_handles[j].wait()
        if j + 1 < nrows:
            nb = (j + 1) % len(bufs)
            if pending_out[nb] is not None:
                pending_out[nb].wait()
                pending_out[nb] = None
            in_handles[j + 1] = pltpu.make_async_copy(
                x_hbm.at[base + (j + 1)], bufs[nb], sems[nb])
            in_handles[j + 1].start()
        process(bufs[b])
        pending_out[b] = pltpu.make_async_copy(
            bufs[b], o_hbm.at[base + j], sems[b])
        pending_out[b].start()
    for h in pending_out:
        if h is not None:
            h.wait()


def kernel(x):
    mesh = plsc.VectorSubcoreMesh(core_axis_name="c", subcore_axis_name="s")
    f = functools.partial(
        pl.kernel,
        out_type=jax.ShapeDtypeStruct((_ROWS, _N), jnp.float32),
        mesh=mesh,
        compiler_params=pltpu.CompilerParams(needs_layout_passes=False),
        scratch_types=[
            pltpu.VMEM((_N,), jnp.float32),
            pltpu.VMEM((_N,), jnp.float32),
            pltpu.VMEM((_N,), jnp.float32),
            pltpu.VMEM((_NBINS,), jnp.int32),
            pltpu.VMEM((_NBINS,), jnp.int32),
            pltpu.SemaphoreType.DMA,
            pltpu.SemaphoreType.DMA,
            pltpu.SemaphoreType.DMA,
        ],
    )(_sc_body)
    return f(x)


# speedup vs baseline: 1.1496x; 1.0542x over previous
import functools

import jax
import jax.numpy as jnp
from jax import lax
from jax.experimental import pallas as pl
from jax.experimental.pallas import tpu as pltpu
from jax.experimental.pallas import tpu_sc as plsc

_K = 64
_L = 16
_NBINS = 256
_ROWS = 128
_N = 32768
_NCHUNK = _N // _L
_UNROLL = 8

_SIGN = -2147483648
_M31 = 0x7FFFFFFF


def _ukey(v, c31):
    b = plsc.bitcast(v, jnp.int32)
    return b ^ (lax.shift_right_arithmetic(b, c31) | jnp.int32(_SIGN))


def _key_to_f32(t_s16):
    bits = jnp.where(t_s16 < 0, t_s16 ^ jnp.int32(_M31), t_s16)
    return plsc.bitcast(bits, jnp.float32)


def _sc_body(x_hbm, o_hbm, xv0, xv1, xv2, hist, fine, sufb,
             sem0, sem1, sem2):
    nc = 2
    wid = lax.axis_index("s") * nc + lax.axis_index("c")
    bufs = (xv0, xv1, xv2)
    sems = (sem0, sem1, sem2)
    lane = lax.iota(jnp.int32, _L)
    c31 = jnp.full((_L,), 31, jnp.int32)
    ones = jnp.ones((_L,), jnp.int32)
    zeros16i = jnp.zeros((_L,), jnp.int32)
    zeros16f = jnp.zeros((_L,), jnp.float32)
    rev_lane = jnp.int32(_L - 1) - lane
    shift_vecs = {s: jnp.full((_L,), s, jnp.int32)
                  for s in (8, 12, 20, 24)}

    def scan256(need):
        def scan(i, carry):
            run, dstar = carry
            cc = (_NBINS // _L - 1) - i
            vv = hist[pl.ds(cc * _L, _L)]
            rv = lax.rev(vv, dimensions=(0,))
            sfx = plsc.cumsum(rv) + run
            sufb[pl.ds(i * _L, _L)] = sfx
            ge = sfx >= need
            pc = plsc.all_reduce_population_count(ge)
            dstar = jnp.maximum(
                dstar, jnp.where(pc > 0, jnp.int32(cc * _L) + pc - 1,
                                 jnp.int32(-1)))
            return jnp.max(sfx), dstar

        _, dstar = lax.fori_loop(
            0, _NBINS // _L, scan,
            (jnp.int32(0), jnp.full((_L,), -1, jnp.int32)))

        sge = plsc.load_gather(sufb, [jnp.int32(_NBINS - 1) - dstar])
        top = dstar >= jnp.int32(_NBINS - 1)
        idx_gt = jnp.where(top, jnp.int32(0), jnp.int32(_NBINS - 2) - dstar)
        cgt = jnp.where(top, jnp.int32(0),
                        plsc.load_gather(sufb, [idx_gt]))
        return dstar, cgt, sge

    def zero_bins():
        @plsc.parallel_loop(0, 4096 // _L, 1, unroll=8)
        def _zf(i):
            fine[pl.ds(i * _L, _L)] = zeros16i

        for z in range(_NBINS // _L):
            hist[pl.ds(z * _L, _L)] = zeros16i

    def run_round12(xv, rnd, prefix, need):
        zero_bins()

        if rnd == 0:
            @plsc.parallel_loop(0, _NCHUNK, 1, unroll=_UNROLL)
            def _hpass(i):
                v = xv[pl.ds(i * _L, _L)]
                u = _ukey(v, c31)
                d12 = lax.shift_right_logical(u, shift_vecs[20])
                d8 = lax.shift_right_logical(u, shift_vecs[24])
                plsc.addupdate_scatter(fine, [d12], ones)
                plsc.addupdate_scatter(hist, [d8], ones)
        else:
            @plsc.parallel_loop(0, _NCHUNK, 1, unroll=_UNROLL)
            def _hpass(i):
                v = xv[pl.ds(i * _L, _L)]
                u = _ukey(v, c31)
                m = lax.shift_right_logical(u, shift_vecs[20]) == prefix
                d12 = (lax.shift_right_logical(u, shift_vecs[8])
                       & jnp.int32(0xFFF))
                d8 = (lax.shift_right_logical(u, shift_vecs[12])
                      & jnp.int32(0xFF))
                plsc.addupdate_scatter(fine, [d12], ones, mask=m)
                plsc.addupdate_scatter(hist, [d8], ones, mask=m)

        c8, cgt_c, sge_c = scan256(need)
        c8s = jnp.max(c8)
        vv = fine[pl.ds(c8s * _L, _L)]
        rv = lax.rev(vv, dimensions=(0,))
        sfx = plsc.cumsum(rv) + cgt_c
        sufb[pl.ds(0, _L)] = sfx
        ge = sfx >= need
        pc = plsc.all_reduce_population_count(ge)
        dstar12 = c8 * jnp.int32(_L) + pc - 1
        l0 = jnp.int32(_L) - pc
        sge = plsc.load_gather(sufb, [l0])
        cgt = jnp.where(pc >= _L, cgt_c,
                        plsc.load_gather(sufb,
                                         [jnp.maximum(l0 - 1, 0)]))
        return dstar12, cgt, sge

    def run_round8(xv, prefix24, need):
        for z in range(_NBINS // _L):
            hist[pl.ds(z * _L, _L)] = zeros16i

        @plsc.parallel_loop(0, _NCHUNK, 1, unroll=_UNROLL)
        def _hpass(i):
            v = xv[pl.ds(i * _L, _L)]
            u = _ukey(v, c31)
            m = lax.shift_right_logical(u, shift_vecs[8]) == prefix24
            dig = u & jnp.int32(0xFF)
            plsc.addupdate_scatter(hist, [dig], ones, mask=m)

        return scan256(need)

    def process(xv):
        k16 = jnp.full((_L,), _K, jnp.int32)
        d0, cgt0, sge0 = run_round12(xv, 0, None, k16)
        p1 = d0
        need1 = k16 - cgt0
        m1 = sge0 - cgt0

        def exit0(_o):
            return p1 * jnp.int32(1 << 20), need1, need1

        def cont0(_o):
            d1, cgt1, sge1 = run_round12(xv, 1, p1, need1)
            p2 = p1 * jnp.int32(4096) + d1
            need2 = need1 - cgt1
            m2 = sge1 - cgt1

            def exit1(_o2):
                return p2 * jnp.int32(1 << 8), need2, need2

            def cont1(_o2):
                d2, cgt2, sge2 = run_round8(xv, p2, need2)
                p3 = p2 * jnp.int32(_NBINS) + d2
                need3 = need2 - cgt2
                m3 = sge2 - cgt2
                return p3, need3, m3

            return lax.cond(jnp.all(m2 == need2), exit1, cont1, 0)

        t_u, r, total_eq = lax.cond(jnp.all(m1 == need1), exit0, cont0, 0)
        t_s = t_u ^ jnp.int32(_SIGN)
        tf = _key_to_f32(t_s)

        def simple_pass(_o):
            @plsc.parallel_loop(0, _NCHUNK, 1, unroll=_UNROLL)
            def _body(i):
                v = xv[pl.ds(i * _L, _L)]
                xv[pl.ds(i * _L, _L)] = jnp.where(v >= tf, v, zeros16f)
            return 0

        def tie_pass(_o):
            def body(i, carry):
                v = xv[pl.ds(i * _L, _L)]
                gt = v > tf
                eq = v == tf
                pc = plsc.cumsum(eq.astype(jnp.int32)) + carry
                keep = gt | (eq & (pc <= r))
                xv[pl.ds(i * _L, _L)] = jnp.where(keep, v, zeros16f)
                return jnp.max(pc)
            lax.fori_loop(0, _NCHUNK, body, jnp.int32(0))
            return 0

        lax.cond(jnp.all(total_eq == r), simple_pass, tie_pass, 0)

    nrows = _ROWS // 32
    base = wid * nrows
    in_handles = [None] * nrows
    pending_out = [None] * len(bufs)

    in_handles[0] = pltpu.make_async_copy(x_hbm.at[base], bufs[0], sems[0])
    in_handles[0].start()
    for j in range(nrows):
        b = j % len(bufs)
        in_handles[j].wait()
        if j + 1 < nrows:
            nb = (j + 1) % len(bufs)
            if pending_out[nb] is not None:
                pending_out[nb].wait()
                pending_out[nb] = None
            in_handles[j + 1] = pltpu.make_async_copy(
                x_hbm.at[base + (j + 1)], bufs[nb], sems[nb])
            in_handles[j + 1].start()
        process(bufs[b])
        pending_out[b] = pltpu.make_async_copy(
            bufs[b], o_hbm.at[base + j], sems[b])
        pending_out[b].start()
    for h in pending_out:
        if h is not None:
            h.wait()


def kernel(x):
    mesh = plsc.VectorSubcoreMesh(core_axis_name="c", subcore_axis_name="s")
    f = functools.partial(
        pl.kernel,
        out_type=jax.ShapeDtypeStruct((_ROWS, _N), jnp.float32),
        mesh=mesh,
        compiler_params=pltpu.CompilerParams(needs_layout_passes=False),
        scratch_types=[
            pltpu.VMEM((_N,), jnp.float32),
            pltpu.VMEM((_N,), jnp.float32),
            pltpu.VMEM((_N,), jnp.float32),
            pltpu.VMEM((_NBINS,), jnp.int32),
            pltpu.VMEM((4096,), jnp.int32),
            pltpu.VMEM((_NBINS,), jnp.int32),
            pltpu.SemaphoreType.DMA,
            pltpu.SemaphoreType.DMA,
            pltpu.SemaphoreType.DMA,
        ],
    )(_sc_body)
    return f(x)


# speedup vs baseline: 1.1571x; 1.0065x over previous
import functools

import jax
import jax.numpy as jnp
from jax import lax
from jax.experimental import pallas as pl
from jax.experimental.pallas import tpu as pltpu
from jax.experimental.pallas import tpu_sc as plsc

_K = 64
_L = 16
_NBINS = 256
_ROWS = 128
_N = 32768
_NCHUNK = _N // _L
_UNROLL = 8

_SIGN = -2147483648
_M31 = 0x7FFFFFFF


def _ukey(v, c31):
    b = plsc.bitcast(v, jnp.int32)
    return b ^ (lax.shift_right_arithmetic(b, c31) | jnp.int32(_SIGN))


def _key_to_f32(t_s16):
    bits = jnp.where(t_s16 < 0, t_s16 ^ jnp.int32(_M31), t_s16)
    return plsc.bitcast(bits, jnp.float32)


def _sc_body(x_hbm, o_hbm, xv0, xv1, xv2, hist, fine, sufb,
             sem0, sem1, sem2):
    nc = 2
    wid = lax.axis_index("s") * nc + lax.axis_index("c")
    bufs = (xv0, xv1, xv2)
    sems = (sem0, sem1, sem2)
    lane = lax.iota(jnp.int32, _L)
    c31 = jnp.full((_L,), 31, jnp.int32)
    ones = jnp.ones((_L,), jnp.int32)
    zeros16i = jnp.zeros((_L,), jnp.int32)
    zeros16f = jnp.zeros((_L,), jnp.float32)
    rev_lane = jnp.int32(_L - 1) - lane
    shift_vecs = {s: jnp.full((_L,), s, jnp.int32)
                  for s in (4, 8, 12, 20, 24)}

    def scan256(need):
        def scan(i, carry):
            run, dstar = carry
            cc = (_NBINS // _L - 1) - i
            vv = hist[pl.ds(cc * _L, _L)]
            rv = lax.rev(vv, dimensions=(0,))
            sfx = plsc.cumsum(rv) + run
            sufb[pl.ds(i * _L, _L)] = sfx
            ge = sfx >= need
            pc = plsc.all_reduce_population_count(ge)
            dstar = jnp.maximum(
                dstar, jnp.where(pc > 0, jnp.int32(cc * _L) + pc - 1,
                                 jnp.int32(-1)))
            return jnp.max(sfx), dstar

        _, dstar = lax.fori_loop(
            0, _NBINS // _L, scan,
            (jnp.int32(0), jnp.full((_L,), -1, jnp.int32)))

        sge = plsc.load_gather(sufb, [jnp.int32(_NBINS - 1) - dstar])
        top = dstar >= jnp.int32(_NBINS - 1)
        idx_gt = jnp.where(top, jnp.int32(0), jnp.int32(_NBINS - 2) - dstar)
        cgt = jnp.where(top, jnp.int32(0),
                        plsc.load_gather(sufb, [idx_gt]))
        return dstar, cgt, sge

    def zero_bins():
        @plsc.parallel_loop(0, 4096 // _L, 1, unroll=8)
        def _zf(i):
            fine[pl.ds(i * _L, _L)] = zeros16i

        for z in range(_NBINS // _L):
            hist[pl.ds(z * _L, _L)] = zeros16i

    def round0_scatter(xv):
        @plsc.parallel_loop(0, _NCHUNK, 1, unroll=_UNROLL)
        def _hpass(i):
            v = xv[pl.ds(i * _L, _L)]
            u = _ukey(v, c31)
            d12 = lax.shift_right_logical(u, shift_vecs[20])
            d8 = lax.shift_right_logical(u, shift_vecs[24])
            plsc.addupdate_scatter(fine, [d12], ones)
            plsc.addupdate_scatter(hist, [d8], ones)

    def run_round12_masked(xv, prefix, need):
        zero_bins()

        @plsc.parallel_loop(0, _NCHUNK, 1, unroll=_UNROLL)
        def _hpass(i):
            v = xv[pl.ds(i * _L, _L)]
            u = _ukey(v, c31)
            m = lax.shift_right_logical(u, shift_vecs[20]) == prefix
            w = (lax.shift_right_logical(u, shift_vecs[8])
                 & jnp.int32(0xFFF))
            d8 = lax.shift_right_logical(w, shift_vecs[4])
            plsc.addupdate_scatter(fine, [w], ones, mask=m)
            plsc.addupdate_scatter(hist, [d8], ones, mask=m)

        return round12_finish(need)

    def round12_finish(need):
        c8, cgt_c, sge_c = scan256(need)
        c8s = jnp.max(c8)
        vv = fine[pl.ds(c8s * _L, _L)]
        rv = lax.rev(vv, dimensions=(0,))
        sfx = plsc.cumsum(rv) + cgt_c
        sufb[pl.ds(0, _L)] = sfx
        ge = sfx >= need
        pc = plsc.all_reduce_population_count(ge)
        dstar12 = c8 * jnp.int32(_L) + pc - 1
        l0 = jnp.int32(_L) - pc
        sge = plsc.load_gather(sufb, [l0])
        cgt = jnp.where(pc >= _L, cgt_c,
                        plsc.load_gather(sufb,
                                         [jnp.maximum(l0 - 1, 0)]))
        return dstar12, cgt, sge

    def run_round8(xv, prefix24, need):
        for z in range(_NBINS // _L):
            hist[pl.ds(z * _L, _L)] = zeros16i

        @plsc.parallel_loop(0, _NCHUNK, 1, unroll=_UNROLL)
        def _hpass(i):
            v = xv[pl.ds(i * _L, _L)]
            u = _ukey(v, c31)
            m = lax.shift_right_logical(u, shift_vecs[8]) == prefix24
            dig = u & jnp.int32(0xFF)
            plsc.addupdate_scatter(hist, [dig], ones, mask=m)

        return scan256(need)

    def select_tail(xv):
        k16 = jnp.full((_L,), _K, jnp.int32)
        d0, cgt0, sge0 = round12_finish(k16)
        p1 = d0
        need1 = k16 - cgt0
        m1 = sge0 - cgt0

        def exit0(_o):
            return p1 * jnp.int32(1 << 20), need1, need1

        def cont0(_o):
            d1, cgt1, sge1 = run_round12_masked(xv, p1, need1)
            p2 = p1 * jnp.int32(4096) + d1
            need2 = need1 - cgt1
            m2 = sge1 - cgt1

            def exit1(_o2):
                return p2 * jnp.int32(1 << 8), need2, need2

            def cont1(_o2):
                d2, cgt2, sge2 = run_round8(xv, p2, need2)
                p3 = p2 * jnp.int32(_NBINS) + d2
                need3 = need2 - cgt2
                m3 = sge2 - cgt2
                return p3, need3, m3

            return lax.cond(jnp.all(m2 == need2), exit1, cont1, 0)

        t_u, r, total_eq = lax.cond(jnp.all(m1 == need1), exit0, cont0, 0)
        t_s = t_u ^ jnp.int32(_SIGN)
        return _key_to_f32(t_s), r, total_eq

    def final_pass(xv, tf, r, total_eq, nxt):
        def simple_pass(_o):
            if nxt is None:
                @plsc.parallel_loop(0, _NCHUNK, 1, unroll=_UNROLL)
                def _body(i):
                    v = xv[pl.ds(i * _L, _L)]
                    xv[pl.ds(i * _L, _L)] = jnp.where(v >= tf, v, zeros16f)
            else:
                @plsc.parallel_loop(0, _NCHUNK, 1, unroll=_UNROLL)
                def _body(i):
                    v = xv[pl.ds(i * _L, _L)]
                    xv[pl.ds(i * _L, _L)] = jnp.where(v >= tf, v, zeros16f)
                    w = nxt[pl.ds(i * _L, _L)]
                    u = _ukey(w, c31)
                    d12 = lax.shift_right_logical(u, shift_vecs[20])
                    d8 = lax.shift_right_logical(u, shift_vecs[24])
                    plsc.addupdate_scatter(fine, [d12], ones)
                    plsc.addupdate_scatter(hist, [d8], ones)
            return 0

        def tie_pass(_o):
            def body(i, carry):
                v = xv[pl.ds(i * _L, _L)]
                gt = v > tf
                eq = v == tf
                pc = plsc.cumsum(eq.astype(jnp.int32)) + carry
                keep = gt | (eq & (pc <= r))
                xv[pl.ds(i * _L, _L)] = jnp.where(keep, v, zeros16f)
                return jnp.max(pc)
            lax.fori_loop(0, _NCHUNK, body, jnp.int32(0))
            if nxt is not None:
                round0_scatter(nxt)
            return 0

        zero_bins()
        lax.cond(jnp.all(total_eq == r), simple_pass, tie_pass, 0)

    nrows = _ROWS // 32
    base = wid * nrows
    in_handles = [None] * nrows
    pending_out = [None] * len(bufs)

    in_handles[0] = pltpu.make_async_copy(x_hbm.at[base], bufs[0], sems[0])
    in_handles[0].start()
    in_handles[0].wait()
    if nrows > 1:
        in_handles[1] = pltpu.make_async_copy(
            x_hbm.at[base + 1], bufs[1], sems[1])
        in_handles[1].start()
    zero_bins()
    round0_scatter(bufs[0])
    for j in range(nrows):
        b = j % len(bufs)
        tf, r, total_eq = select_tail(bufs[b])
        if j + 1 < nrows:
            nb = (j + 1) % len(bufs)
            in_handles[j + 1].wait()
            final_pass(bufs[b], tf, r, total_eq, bufs[nb])
        else:
            final_pass(bufs[b], tf, r, total_eq, None)
        pending_out[b] = pltpu.make_async_copy(
            bufs[b], o_hbm.at[base + j], sems[b])
        pending_out[b].start()
        if j + 2 < nrows:
            nnb = (j + 2) % len(bufs)
            if pending_out[nnb] is not None:
                pending_out[nnb].wait()
                pending_out[nnb] = None
            in_handles[j + 2] = pltpu.make_async_copy(
                x_hbm.at[base + (j + 2)], bufs[nnb], sems[nnb])
            in_handles[j + 2].start()
    for h in pending_out:
        if h is not None:
            h.wait()


def kernel(x):
    mesh = plsc.VectorSubcoreMesh(core_axis_name="c", subcore_axis_name="s")
    f = functools.partial(
        pl.kernel,
        out_type=jax.ShapeDtypeStruct((_ROWS, _N), jnp.float32),
        mesh=mesh,
        compiler_params=pltpu.CompilerParams(needs_layout_passes=False),
        scratch_types=[
            pltpu.VMEM((_N,), jnp.float32),
            pltpu.VMEM((_N,), jnp.float32),
            pltpu.VMEM((_N,), jnp.float32),
            pltpu.VMEM((_NBINS,), jnp.int32),
            pltpu.VMEM((4096,), jnp.int32),
            pltpu.VMEM((_NBINS,), jnp.int32),
            pltpu.SemaphoreType.DMA,
            pltpu.SemaphoreType.DMA,
            pltpu.SemaphoreType.DMA,
        ],
    )(_sc_body)
    return f(x)
